# Initial kernel scaffold; baseline (speedup 1.0000x reference)
import jax
import jax.numpy as jnp
from jax.experimental import pallas as pl


def kernel(x, table):
    raise NotImplementedError("write your pallas kernel here")


# speedup vs baseline: 3.3421x; 3.3421x over previous
import functools

import jax
import jax.numpy as jnp
from jax import lax
from jax.experimental import pallas as pl
from jax.experimental.pallas import tpu as pltpu
from jax.experimental.pallas import tpu_sc as plsc

_NC = 2
_NS = 16
_NW = _NC * _NS
_C = 128
_NBUF = 5


@functools.lru_cache(maxsize=None)
def _make_gather(B, D):
    per_w = B // _NW
    nchunk = per_w // _C
    nblk = nchunk // _NBUF
    mesh = plsc.VectorSubcoreMesh(core_axis_name="c", subcore_axis_name="s")

    scratch = [pltpu.VMEM((nchunk, _C), jnp.int32)]
    scratch += [pltpu.VMEM((_C, D), jnp.float32) for _ in range(_NBUF)]
    scratch += [pltpu.SemaphoreType.DMA for _ in range(_NBUF)]

    @functools.partial(
        pl.kernel,
        mesh=mesh,
        out_type=jax.ShapeDtypeStruct((B, D), jnp.float32),
        scratch_types=scratch,
    )
    def k(idx_hbm, table_hbm, out_hbm, idx_v, *rest):
        bufs = rest[:_NBUF]
        sems = rest[_NBUF:]
        wid = lax.axis_index("s") * _NC + lax.axis_index("c")
        row0 = wid * per_w
        pltpu.sync_copy(idx_hbm.at[wid], idx_v)

        def start_gather(g, b):
            pltpu.make_async_copy(table_hbm.at[idx_v.at[g]], bufs[b], sems[b]).start()

        def wait_gather(g, b):
            pltpu.make_async_copy(table_hbm.at[idx_v.at[g]], bufs[b], sems[b]).wait()

        for b in range(_NBUF):
            start_gather(b, b)

        def body(blk, carry):
            for b in range(_NBUF):
                g = blk * _NBUF + b
                wait_gather(g, b)
                pltpu.sync_copy(bufs[b], out_hbm.at[pl.ds(row0 + g * _C, _C)])
                start_gather(g + _NBUF, b)
            return carry

        lax.fori_loop(0, nblk - 1, body, 0)

        for b in range(_NBUF):
            g = (nblk - 1) * _NBUF + b
            wait_gather(g, b)
            pltpu.sync_copy(bufs[b], out_hbm.at[pl.ds(row0 + g * _C, _C)])

    return k


def kernel(x, table):
    B = x.size
    D = table.shape[1]
    idx = x.reshape(_NW, (B // _NW) // _C, _C).astype(jnp.int32)
    out = _make_gather(B, D)(idx, table)
    return out.reshape(x.shape + (D,))


# speedup vs baseline: 5.9625x; 1.7840x over previous
import functools

import jax
import jax.numpy as jnp
from jax import lax
from jax.experimental import pallas as pl
from jax.experimental.pallas import tpu as pltpu
from jax.experimental.pallas import tpu_sc as plsc

_NC = 2
_NS = 16
_NW = _NC * _NS
_NBUF = 8


@functools.lru_cache(maxsize=None)
def _make_gather(N, S, D):
    per_w = N // _NW
    nblk = per_w // _NBUF
    mesh = plsc.VectorSubcoreMesh(core_axis_name="c", subcore_axis_name="s")

    scratch = [pltpu.VMEM((per_w, S), jnp.int32)]
    scratch += [pltpu.VMEM((S, D), jnp.float32) for _ in range(_NBUF)]
    scratch += [pltpu.SemaphoreType.DMA for _ in range(_NBUF)]

    @functools.partial(
        pl.kernel,
        mesh=mesh,
        out_type=jax.ShapeDtypeStruct((N, S, D), jnp.float32),
        scratch_types=scratch,
    )
    def k(idx_hbm, table_hbm, out_hbm, idx_v, *rest):
        bufs = rest[:_NBUF]
        sems = rest[_NBUF:]
        wid = lax.axis_index("s") * _NC + lax.axis_index("c")
        b0 = wid * per_w
        pltpu.sync_copy(idx_hbm.at[wid], idx_v)

        def start_gather(g, b):
            pltpu.make_async_copy(table_hbm.at[idx_v.at[g]], bufs[b], sems[b]).start()

        def wait_gather(g, b):
            pltpu.make_async_copy(table_hbm.at[idx_v.at[g]], bufs[b], sems[b]).wait()

        for b in range(_NBUF):
            start_gather(b, b)

        def body(blk, carry):
            for b in range(_NBUF):
                g = blk * _NBUF + b
                wait_gather(g, b)
                pltpu.sync_copy(bufs[b], out_hbm.at[b0 + g])
                start_gather(g + _NBUF, b)
            return carry

        lax.fori_loop(0, nblk - 1, body, 0)

        for b in range(_NBUF):
            g = (nblk - 1) * _NBUF + b
            wait_gather(g, b)
            pltpu.sync_copy(bufs[b], out_hbm.at[b0 + g])

    return k


def kernel(x, table):
    N, S = x.shape
    D = table.shape[1]
    idx = x.reshape(_NW, N // _NW, S).astype(jnp.int32)
    return _make_gather(N, S, D)(idx, table)
